# baseline (speedup 1.0000x reference)
import functools

import jax
import jax.numpy as jnp
from jax import lax
from jax.experimental import pallas as pl
from jax.experimental.pallas import tpu as pltpu
from jax.experimental.pallas import tpu_sc as plsc

D = 32
F = 32
B = 16384

_NC, _NS = 2, 16
_NW = _NC * _NS
_BPW = B // _NW


def _sc_gather(uids, iids, uemb, iemb, ult, ilt,
               out_u, out_i, out_ult, out_ilt,
               uidx_v, iidx_v, urows_v, irows_v, ultv, iltv, sem):
    wid = lax.axis_index("s") * _NC + lax.axis_index("c")
    base = wid * _BPW
    pltpu.sync_copy(uids.at[pl.ds(base, _BPW)], uidx_v)
    pltpu.sync_copy(iids.at[pl.ds(base, _BPW)], iidx_v)
    c1 = pltpu.async_copy(uemb.at[uidx_v], urows_v, sem)
    c2 = pltpu.async_copy(iemb.at[iidx_v], irows_v, sem)
    c3 = pltpu.async_copy(ult.at[uidx_v], ultv, sem)
    c4 = pltpu.async_copy(ilt.at[iidx_v], iltv, sem)
    c1.wait()
    c2.wait()
    c3.wait()
    c4.wait()
    pltpu.sync_copy(urows_v, out_u.at[pl.ds(base, _BPW)])
    pltpu.sync_copy(irows_v, out_i.at[pl.ds(base, _BPW)])
    pltpu.sync_copy(ultv, out_ult.at[pl.ds(base, _BPW)])
    pltpu.sync_copy(iltv, out_ilt.at[pl.ds(base, _BPW)])


_gather_call = functools.partial(
    pl.kernel,
    mesh=plsc.VectorSubcoreMesh(core_axis_name="c", subcore_axis_name="s",
                                num_cores=_NC, num_subcores=_NS),
    out_type=[
        jax.ShapeDtypeStruct((B, D), jnp.float32),
        jax.ShapeDtypeStruct((B, D), jnp.float32),
        jax.ShapeDtypeStruct((B,), jnp.float32),
        jax.ShapeDtypeStruct((B,), jnp.float32),
    ],
    scratch_types=[
        pltpu.VMEM((_BPW,), jnp.int32),
        pltpu.VMEM((_BPW,), jnp.int32),
        pltpu.VMEM((_BPW, D), jnp.float32),
        pltpu.VMEM((_BPW, D), jnp.float32),
        pltpu.VMEM((_BPW,), jnp.float32),
        pltpu.VMEM((_BPW,), jnp.float32),
        pltpu.SemaphoreType.DMA,
    ],
    compiler_params=pltpu.CompilerParams(use_tc_tiling_on_sc=False),
)(_sc_gather)

_BB = 2048


def _dense_body(qt_ref, ts_ref, ult_ref, ilt_ref, u_ref, i_ref, f_ref,
                tpw_ref, w_ref, b_ref, nu_ref, ni_ref, pred_ref):
    tpw = tpw_ref[...]
    ts = ts_ref[...]
    qt = qt_ref[0, 0]
    u = u_ref[...]
    it = i_ref[...]
    us = u * (1.0 + (ts - ult_ref[...]) * tpw)
    isc = it * (1.0 + (ts - ilt_ref[...]) * tpw)
    ps = u * (1.0 + (qt - ult_ref[...]) * tpw)
    z = jnp.concatenate([us, isc, f_ref[...], ps], axis=1)
    out = jnp.dot(z, w_ref[...], preferred_element_type=jnp.float32,
                  precision=jax.lax.Precision.HIGHEST)
    out = out + b_ref[...]
    nu_ref[...] = jnp.tanh(out[:, :D])
    ni_ref[...] = jnp.tanh(out[:, D:2 * D])
    pred_ref[...] = out[:, 2 * D:3 * D]


def _dense_call(qt, ts, ultc, iltc, urows, irows, features, tpw_row, w_all, b_all):
    grid = (B // _BB,)
    col = lambda ib: (ib, 0)
    fixed = lambda ib: (0, 0)
    return pl.pallas_call(
        _dense_body,
        grid=grid,
        in_specs=[
            pl.BlockSpec(memory_space=pltpu.SMEM),
            pl.BlockSpec((_BB, 1), col),
            pl.BlockSpec((_BB, 1), col),
            pl.BlockSpec((_BB, 1), col),
            pl.BlockSpec((_BB, D), col),
            pl.BlockSpec((_BB, D), col),
            pl.BlockSpec((_BB, F), col),
            pl.BlockSpec((1, D), fixed),
            pl.BlockSpec((4 * D, 3 * D), fixed),
            pl.BlockSpec((1, 3 * D), fixed),
        ],
        out_specs=[
            pl.BlockSpec((_BB, D), col),
            pl.BlockSpec((_BB, D), col),
            pl.BlockSpec((_BB, D), col),
        ],
        out_shape=[
            jax.ShapeDtypeStruct((B, D), jnp.float32),
            jax.ShapeDtypeStruct((B, D), jnp.float32),
            jax.ShapeDtypeStruct((B, D), jnp.float32),
        ],
        compiler_params=pltpu.CompilerParams(
            dimension_semantics=("arbitrary",)),
    )(qt, ts, ultc, iltc, urows, irows, features, tpw_row, w_all, b_all)


def kernel(user_ids, item_ids, timestamps, features, query_time,
           user_embeddings, item_embeddings, user_last_time, item_last_time,
           time_proj_w, Wp, bp,
           W_ih_u, W_hh_u, b_ih_u, b_hh_u,
           W_ih_i, W_hh_i, b_ih_i, b_hh_i):
    urows, irows, ult, ilt = _gather_call(
        user_ids, item_ids, user_embeddings, item_embeddings,
        user_last_time, item_last_time)

    zeros = jnp.zeros((D, D), jnp.float32)
    wu = jnp.concatenate([W_ih_u[:, :D].T + W_hh_u.T,
                          W_ih_u[:, D:2 * D].T,
                          W_ih_u[:, 2 * D:].T,
                          zeros], axis=0)
    wi = jnp.concatenate([W_ih_i[:, D:2 * D].T,
                          W_ih_i[:, :D].T + W_hh_i.T,
                          W_ih_i[:, 2 * D:].T,
                          zeros], axis=0)
    wp = jnp.concatenate([zeros, zeros, zeros, Wp.T], axis=0)
    w_all = jnp.concatenate([wu, wi, wp], axis=1)
    b_all = jnp.concatenate([b_ih_u + b_hh_u, b_ih_i + b_hh_i, bp])[None, :]

    qt = jnp.full((1, 1), query_time, jnp.float32)
    ts = timestamps[:, None]
    tpw_row = time_proj_w.T

    nu, ni, pred = _dense_call(qt, ts, ult[:, None], ilt[:, None],
                               urows, irows, features, tpw_row, w_all, b_all)
    return (pred, nu, ni)
